# Initial kernel scaffold; baseline (speedup 1.0000x reference)
import jax
import jax.numpy as jnp
from jax.experimental import pallas as pl


def kernel(features, edge_index, W_mp, b_mp, W1, b1, W2, b2):
    raise NotImplementedError("write your pallas kernel here")


# speedup vs baseline: 5.7130x; 5.7130x over previous
import functools

import jax
import jax.numpy as jnp
from jax import lax
from jax.experimental import pallas as pl
from jax.experimental.pallas import tpu as pltpu
from jax.experimental.pallas import tpu_sc as plsc

N = 10000
E = 320000
D = 128
DX = 144
NC = 2
NS = 16
NW = NC * NS
EW = E // NW
C = 80
IT = EW // C
ZCH = 80
NCH = N // ZCH
TZ = -(-NCH // NS)


def _widen_mm_body(f_ref, w_ref, o_ref):
    mm = jnp.dot(f_ref[...], w_ref[...], preferred_element_type=jnp.float32)
    o_ref[...] = jnp.concatenate(
        [mm, jnp.ones((N, DX - D), jnp.float32)], axis=1)


def _head_body(p_ref, bmp_ref, w1_ref, b1_ref, w2_ref, b2_ref, o_ref):
    agg = p_ref[0, :, :D] + p_ref[1, :, :D]
    degw = p_ref[0, :, D:DX] + p_ref[1, :, D:DX]
    inv = 1.0 / jnp.maximum(degw[:, :1], 1.0)
    h = jnp.maximum(agg * inv + bmp_ref[...], 0.0)
    h = jnp.maximum(
        jnp.dot(h, w1_ref[...], preferred_element_type=jnp.float32)
        + b1_ref[...], 0.0)
    o_ref[...] = (
        jnp.dot(h, w2_ref[...], preferred_element_type=jnp.float32)
        + b2_ref[...])


def _sc_body(ftx_hbm, src_hbm, dst_hbm, out_hbm, sidx, didx, rows, zbuf,
             acc, sem):
    cid = lax.axis_index("c")
    sid = lax.axis_index("s")
    w = cid * NS + sid

    def zrow(r, carry):
        for c9 in range(DX // 16):
            zbuf[r, pl.ds(c9 * 16, 16)] = jnp.zeros((16,), jnp.float32)
        return carry
    lax.fori_loop(0, ZCH, zrow, 0)

    def zchunk(t, carry):
        j = t * NS + sid

        @pl.when(j < NCH)
        def _():
            r0 = pl.multiple_of(j * ZCH, 8)
            pltpu.sync_copy(zbuf, acc.at[pl.ds(r0, ZCH)])
        return carry
    lax.fori_loop(0, TZ, zchunk, 0)
    plsc.subcore_barrier()

    def step(i, carry):
        base = pl.multiple_of(w * EW + i * C, 8)
        pltpu.sync_copy(src_hbm.at[pl.ds(base, C)], sidx)
        pltpu.sync_copy(dst_hbm.at[pl.ds(base, C)], didx)
        pltpu.async_copy(ftx_hbm.at[sidx], rows, sem).wait()
        pltpu.sync_copy(rows, acc.at[didx], add=True)
        return carry
    lax.fori_loop(0, IT, step, 0)
    plsc.subcore_barrier()

    def wchunk(t, carry):
        j = t * NS + sid

        @pl.when(j < NCH)
        def _():
            r0 = pl.multiple_of(j * ZCH, 8)
            pltpu.sync_copy(acc.at[pl.ds(r0, ZCH)], zbuf)
            pltpu.sync_copy(zbuf, out_hbm.at[cid, pl.ds(r0, ZCH)])
        return carry
    lax.fori_loop(0, TZ, wchunk, 0)


_sc_aggregate = functools.partial(
    pl.kernel,
    out_type=jax.ShapeDtypeStruct((NC, N, DX), jnp.float32),
    mesh=plsc.VectorSubcoreMesh(
        core_axis_name="c", subcore_axis_name="s",
        num_cores=NC, num_subcores=NS),
    scratch_types=[
        pltpu.VMEM((C,), jnp.int32),
        pltpu.VMEM((C,), jnp.int32),
        pltpu.VMEM((C, DX), jnp.float32),
        pltpu.VMEM((ZCH, DX), jnp.float32),
        pltpu.VMEM_SHARED((N, DX), jnp.float32),
        pltpu.SemaphoreType.DMA,
    ],
    compiler_params=pltpu.CompilerParams(use_tc_tiling_on_sc=False),
)(_sc_body)


def kernel(features, edge_index, W_mp, b_mp, W1, b1, W2, b2):
    ftx = pl.pallas_call(
        _widen_mm_body,
        out_shape=jax.ShapeDtypeStruct((N, DX), jnp.float32),
    )(features, W_mp)

    parts = _sc_aggregate(ftx, edge_index[0], edge_index[1])

    out = pl.pallas_call(
        _head_body,
        out_shape=jax.ShapeDtypeStruct((N, D), jnp.float32),
    )(parts, b_mp.reshape(1, D), W1, b1.reshape(1, D), W2, b2.reshape(1, D))
    return out


# speedup vs baseline: 12.7811x; 2.2372x over previous
import functools

import jax
import jax.numpy as jnp
from jax import lax
from jax.experimental import pallas as pl
from jax.experimental.pallas import tpu as pltpu
from jax.experimental.pallas import tpu_sc as plsc

N = 10000
E = 320000
D = 128
DG = 16
NC = 2
NS = 16
NW = NC * NS
EW = E // NW
C = 100
IT = EW // C
P = 2
PC = IT // P
NB = 2
ZA = 100
ZD = 200


def _mm_body(f_ref, w_ref, o_ref):
    o_ref[...] = jnp.dot(f_ref[...], w_ref[...],
                         preferred_element_type=jnp.float32)


def _head_body(p_ref, g_ref, bmp_ref, w1_ref, b1_ref, w2_ref, b2_ref, o_ref):
    agg = p_ref[0] + p_ref[1]
    inv = 1.0 / jnp.maximum(g_ref[0, :, :1] + g_ref[1, :, :1], 1.0)
    h = jnp.maximum(agg * inv + bmp_ref[...], 0.0)
    h = jnp.maximum(
        jnp.dot(h, w1_ref[...], preferred_element_type=jnp.float32)
        + b1_ref[...], 0.0)
    o_ref[...] = (
        jnp.dot(h, w2_ref[...], preferred_element_type=jnp.float32)
        + b2_ref[...])


def _sc_body(ft_hbm, src_hbm, dst_hbm, agg_hbm, deg_hbm,
             sph, dph, r0b, r1b, onesb, zdeg, acc, deg, s0, s1):
    rows = [r0b, r1b]
    sems = [s0, s1]
    cid = lax.axis_index("c")
    sid = lax.axis_index("s")
    w = cid * NS + sid

    def frow(r, carry):
        for c8 in range(D // 16):
            r0b[r, pl.ds(c8 * 16, 16)] = jnp.zeros((16,), jnp.float32)
        onesb[r, pl.ds(0, DG)] = jnp.ones((DG,), jnp.float32)
        return carry
    lax.fori_loop(0, C, frow, 0)

    def fzd(r, carry):
        zdeg[r, pl.ds(0, DG)] = jnp.zeros((DG,), jnp.float32)
        return carry
    lax.fori_loop(0, ZD, fzd, 0)

    def zacc(t, carry):
        j = t * NS + sid

        @pl.when(j < N // ZA)
        def _():
            pltpu.sync_copy(r0b, acc.at[pl.ds(j * ZA, ZA)])
        return carry
    lax.fori_loop(0, -(-(N // ZA) // NS), zacc, 0)

    def zdg(t, carry):
        j = t * NS + sid

        @pl.when(j < N // ZD)
        def _():
            pltpu.sync_copy(zdeg, deg.at[pl.ds(j * ZD, ZD)])
        return carry
    lax.fori_loop(0, -(-(N // ZD) // NS), zdg, 0)
    plsc.subcore_barrier()

    for p in range(P):
        pltpu.sync_copy(src_hbm.at[w, pl.ds(p * PC, PC)], sph)
        pltpu.sync_copy(dst_hbm.at[w, pl.ds(p * PC, PC)], dph)
        for b in range(NB):
            pltpu.async_copy(ft_hbm.at[sph.at[b]], rows[b], sems[b])

        def step(t, carry):
            for b in range(NB):
                i = t * NB + b
                pltpu.make_async_copy(
                    ft_hbm.at[sph.at[i]], rows[b], sems[b]).wait()
                pltpu.sync_copy(rows[b], acc.at[dph.at[i]], add=True)
                pltpu.sync_copy(onesb, deg.at[dph.at[i]], add=True)
                pltpu.async_copy(ft_hbm.at[sph.at[i + NB]], rows[b], sems[b])
            return carry
        lax.fori_loop(0, PC // NB - 1, step, 0)

        for b in range(NB):
            i = PC - NB + b
            pltpu.make_async_copy(
                ft_hbm.at[sph.at[i]], rows[b], sems[b]).wait()
            pltpu.sync_copy(rows[b], acc.at[dph.at[i]], add=True)
            pltpu.sync_copy(onesb, deg.at[dph.at[i]], add=True)
    plsc.subcore_barrier()

    def wacc(t, carry):
        j = t * NS + sid

        @pl.when(j < N // ZA)
        def _():
            pltpu.sync_copy(acc.at[pl.ds(j * ZA, ZA)], r0b)
            pltpu.sync_copy(r0b, agg_hbm.at[cid, pl.ds(j * ZA, ZA)])
        return carry
    lax.fori_loop(0, -(-(N // ZA) // NS), wacc, 0)

    def wdg(t, carry):
        j = t * NS + sid

        @pl.when(j < N // ZD)
        def _():
            pltpu.sync_copy(deg.at[pl.ds(j * ZD, ZD)], zdeg)
            pltpu.sync_copy(zdeg, deg_hbm.at[cid, pl.ds(j * ZD, ZD)])
        return carry
    lax.fori_loop(0, -(-(N // ZD) // NS), wdg, 0)


_sc_aggregate = functools.partial(
    pl.kernel,
    out_type=(jax.ShapeDtypeStruct((NC, N, D), jnp.float32),
              jax.ShapeDtypeStruct((NC, N, DG), jnp.float32)),
    mesh=plsc.VectorSubcoreMesh(
        core_axis_name="c", subcore_axis_name="s",
        num_cores=NC, num_subcores=NS),
    scratch_types=(
        [pltpu.VMEM((PC, C), jnp.int32)] * 2
        + [pltpu.VMEM((C, D), jnp.float32)] * NB
        + [pltpu.VMEM((C, DG), jnp.float32),
           pltpu.VMEM((ZD, DG), jnp.float32),
           pltpu.VMEM_SHARED((N, D), jnp.float32),
           pltpu.VMEM_SHARED((N, DG), jnp.float32)]
        + [pltpu.SemaphoreType.DMA] * NB
    ),
    compiler_params=pltpu.CompilerParams(use_tc_tiling_on_sc=False),
)(_sc_body)


def kernel(features, edge_index, W_mp, b_mp, W1, b1, W2, b2):
    ft = pl.pallas_call(
        _mm_body,
        out_shape=jax.ShapeDtypeStruct((N, D), jnp.float32),
    )(features, W_mp)

    parts, degp = _sc_aggregate(ft,
                                edge_index[0].reshape(NW, IT, C),
                                edge_index[1].reshape(NW, IT, C))

    out = pl.pallas_call(
        _head_body,
        out_shape=jax.ShapeDtypeStruct((N, D), jnp.float32),
    )(parts, degp, b_mp.reshape(1, D), W1, b1.reshape(1, D),
      W2, b2.reshape(1, D))
    return out


# speedup vs baseline: 13.3167x; 1.0419x over previous
import functools

import jax
import jax.numpy as jnp
from jax import lax
from jax.experimental import pallas as pl
from jax.experimental.pallas import tpu as pltpu
from jax.experimental.pallas import tpu_sc as plsc

N = 10000
E = 320000
D = 128
DG = 16
NC = 2
NS = 16
NW = NC * NS
EW = E // NW
C = 50
IT = EW // C
P = 2
PC = IT // P
NB = 4
ZA = C
ZD = 200


def _mm_body(f_ref, w_ref, o_ref):
    o_ref[...] = jnp.dot(f_ref[...], w_ref[...],
                         preferred_element_type=jnp.float32)


def _head_body(p_ref, g_ref, bmp_ref, w1_ref, b1_ref, w2_ref, b2_ref, o_ref):
    agg = p_ref[0] + p_ref[1]
    inv = 1.0 / jnp.maximum(g_ref[0, :, :1] + g_ref[1, :, :1], 1.0)
    h = jnp.maximum(agg * inv + bmp_ref[...], 0.0)
    h = jnp.maximum(
        jnp.dot(h, w1_ref[...], preferred_element_type=jnp.float32)
        + b1_ref[...], 0.0)
    o_ref[...] = (
        jnp.dot(h, w2_ref[...], preferred_element_type=jnp.float32)
        + b2_ref[...])


def _sc_body(ft_hbm, ei_hbm, agg_hbm, deg_hbm,
             sph, dph, r0b, r1b, r2b, r3b, onesb, zdeg, acc, deg,
             s0, s1, s2, s3, sd):
    rows = [r0b, r1b, r2b, r3b]
    sems = [s0, s1, s2, s3]
    cid = lax.axis_index("c")
    sid = lax.axis_index("s")
    w = cid * NS + sid

    def frow(r, carry):
        for c8 in range(D // 16):
            r0b[r, pl.ds(c8 * 16, 16)] = jnp.zeros((16,), jnp.float32)
        onesb[r, pl.ds(0, DG)] = jnp.ones((DG,), jnp.float32)
        return carry
    lax.fori_loop(0, C, frow, 0)

    def fzd(r, carry):
        zdeg[r, pl.ds(0, DG)] = jnp.zeros((DG,), jnp.float32)
        return carry
    lax.fori_loop(0, ZD, fzd, 0)

    def zacc(t, carry):
        j = t * NS + sid

        @pl.when(j < N // ZA)
        def _():
            pltpu.sync_copy(r0b, acc.at[pl.ds(j * ZA, ZA)])
        return carry
    lax.fori_loop(0, -(-(N // ZA) // NS), zacc, 0)

    def zdg(t, carry):
        j = t * NS + sid

        @pl.when(j < N // ZD)
        def _():
            pltpu.sync_copy(zdeg, deg.at[pl.ds(j * ZD, ZD)])
        return carry
    lax.fori_loop(0, -(-(N // ZD) // NS), zdg, 0)
    plsc.subcore_barrier()

    for p in range(P):
        pltpu.sync_copy(ei_hbm.at[0, w, pl.ds(p * PC, PC)], sph)
        pltpu.sync_copy(ei_hbm.at[1, w, pl.ds(p * PC, PC)], dph)
        for b in range(NB):
            pltpu.async_copy(ft_hbm.at[sph.at[b]], rows[b], sems[b])

        def step(t, carry):
            for b in range(NB):
                i = t * NB + b
                pltpu.make_async_copy(
                    ft_hbm.at[sph.at[i]], rows[b], sems[b]).wait()
                pltpu.sync_copy(rows[b], acc.at[dph.at[i]], add=True)
                pltpu.async_copy(onesb, deg.at[dph.at[i]], sd, add=True)
                pltpu.async_copy(ft_hbm.at[sph.at[i + NB]], rows[b], sems[b])
            return carry
        lax.fori_loop(0, PC // NB - 1, step, 0)

        for b in range(NB):
            i = PC - NB + b
            pltpu.make_async_copy(
                ft_hbm.at[sph.at[i]], rows[b], sems[b]).wait()
            pltpu.sync_copy(rows[b], acc.at[dph.at[i]], add=True)
            pltpu.async_copy(onesb, deg.at[dph.at[i]], sd, add=True)

        def drain(i, carry):
            pltpu.make_async_copy(onesb, deg.at[dph.at[i]], sd).wait()
            return carry
        lax.fori_loop(0, PC, drain, 0)
    plsc.subcore_barrier()

    def wacc(t, carry):
        j = t * NS + sid

        @pl.when(j < N // ZA)
        def _():
            pltpu.sync_copy(acc.at[pl.ds(j * ZA, ZA)], r0b)
            pltpu.sync_copy(r0b, agg_hbm.at[cid, pl.ds(j * ZA, ZA)])
        return carry
    lax.fori_loop(0, -(-(N // ZA) // NS), wacc, 0)

    def wdg(t, carry):
        j = t * NS + sid

        @pl.when(j < N // ZD)
        def _():
            pltpu.sync_copy(deg.at[pl.ds(j * ZD, ZD)], zdeg)
            pltpu.sync_copy(zdeg, deg_hbm.at[cid, pl.ds(j * ZD, ZD)])
        return carry
    lax.fori_loop(0, -(-(N // ZD) // NS), wdg, 0)


_sc_aggregate = functools.partial(
    pl.kernel,
    out_type=(jax.ShapeDtypeStruct((NC, N, D), jnp.float32),
              jax.ShapeDtypeStruct((NC, N, DG), jnp.float32)),
    mesh=plsc.VectorSubcoreMesh(
        core_axis_name="c", subcore_axis_name="s",
        num_cores=NC, num_subcores=NS),
    scratch_types=(
        [pltpu.VMEM((PC, C), jnp.int32)] * 2
        + [pltpu.VMEM((C, D), jnp.float32)] * NB
        + [pltpu.VMEM((C, DG), jnp.float32),
           pltpu.VMEM((ZD, DG), jnp.float32),
           pltpu.VMEM_SHARED((N, D), jnp.float32),
           pltpu.VMEM_SHARED((N, DG), jnp.float32)]
        + [pltpu.SemaphoreType.DMA] * (NB + 1)
    ),
    compiler_params=pltpu.CompilerParams(use_tc_tiling_on_sc=False),
)(_sc_body)


def kernel(features, edge_index, W_mp, b_mp, W1, b1, W2, b2):
    ft = pl.pallas_call(
        _mm_body,
        out_shape=jax.ShapeDtypeStruct((N, D), jnp.float32),
    )(features, W_mp)

    parts, degp = _sc_aggregate(ft, edge_index.reshape(2, NW, IT, C))

    out = pl.pallas_call(
        _head_body,
        out_shape=jax.ShapeDtypeStruct((N, D), jnp.float32),
    )(parts, degp, b_mp.reshape(1, D), W1, b1.reshape(1, D),
      W2, b2.reshape(1, D))
    return out


# speedup vs baseline: 13.6447x; 1.0246x over previous
import functools

import jax
import jax.numpy as jnp
from jax import lax
from jax.experimental import pallas as pl
from jax.experimental.pallas import tpu as pltpu
from jax.experimental.pallas import tpu_sc as plsc

N = 10000
E = 320000
D = 128
DG = 16
NC = 2
NS = 16
NW = NC * NS
EW = E // NW
C = 50
IT = EW // C
P = 2
PC = IT // P
NB = 4
ZA = C
ZD = 200


def _head_body(p_ref, g_ref, wmp_ref, bmp_ref, w1_ref, b1_ref, w2_ref,
               b2_ref, o_ref):
    agg = p_ref[0] + p_ref[1]
    inv = 1.0 / jnp.maximum(g_ref[0, :, :1] + g_ref[1, :, :1], 1.0)
    h = jnp.maximum(
        jnp.dot(agg * inv, wmp_ref[...], preferred_element_type=jnp.float32)
        + bmp_ref[...], 0.0)
    h = jnp.maximum(
        jnp.dot(h, w1_ref[...], preferred_element_type=jnp.float32)
        + b1_ref[...], 0.0)
    o_ref[...] = (
        jnp.dot(h, w2_ref[...], preferred_element_type=jnp.float32)
        + b2_ref[...])


def _sc_body(ft_hbm, ei_hbm, agg_hbm, deg_hbm,
             sph, dph, r0b, r1b, r2b, r3b, onesb, zdeg, acc, deg,
             s0, s1, s2, s3, sd):
    rows = [r0b, r1b, r2b, r3b]
    sems = [s0, s1, s2, s3]
    cid = lax.axis_index("c")
    sid = lax.axis_index("s")
    w = cid * NS + sid

    def frow(r, carry):
        for c8 in range(D // 16):
            r0b[r, pl.ds(c8 * 16, 16)] = jnp.zeros((16,), jnp.float32)
        onesb[r, pl.ds(0, DG)] = jnp.ones((DG,), jnp.float32)
        return carry
    lax.fori_loop(0, C, frow, 0)

    def fzd(r, carry):
        zdeg[r, pl.ds(0, DG)] = jnp.zeros((DG,), jnp.float32)
        return carry
    lax.fori_loop(0, ZD, fzd, 0)

    def zacc(t, carry):
        j = t * NS + sid

        @pl.when(j < N // ZA)
        def _():
            pltpu.sync_copy(r0b, acc.at[pl.ds(j * ZA, ZA)])
        return carry
    lax.fori_loop(0, -(-(N // ZA) // NS), zacc, 0)

    def zdg(t, carry):
        j = t * NS + sid

        @pl.when(j < N // ZD)
        def _():
            pltpu.sync_copy(zdeg, deg.at[pl.ds(j * ZD, ZD)])
        return carry
    lax.fori_loop(0, -(-(N // ZD) // NS), zdg, 0)
    plsc.subcore_barrier()

    for p in range(P):
        pltpu.sync_copy(ei_hbm.at[0, w, pl.ds(p * PC, PC)], sph)
        pltpu.sync_copy(ei_hbm.at[1, w, pl.ds(p * PC, PC)], dph)
        for b in range(NB):
            pltpu.async_copy(ft_hbm.at[sph.at[b]], rows[b], sems[b])

        def step(t, carry):
            for b in range(NB):
                i = t * NB + b
                pltpu.make_async_copy(
                    ft_hbm.at[sph.at[i]], rows[b], sems[b]).wait()
                pltpu.sync_copy(rows[b], acc.at[dph.at[i]], add=True)
                pltpu.async_copy(onesb, deg.at[dph.at[i]], sd, add=True)
                pltpu.async_copy(ft_hbm.at[sph.at[i + NB]], rows[b], sems[b])
            return carry
        lax.fori_loop(0, PC // NB - 1, step, 0)

        for b in range(NB):
            i = PC - NB + b
            pltpu.make_async_copy(
                ft_hbm.at[sph.at[i]], rows[b], sems[b]).wait()
            pltpu.sync_copy(rows[b], acc.at[dph.at[i]], add=True)
            pltpu.async_copy(onesb, deg.at[dph.at[i]], sd, add=True)

        def drain(i, carry):
            pltpu.make_async_copy(onesb, deg.at[dph.at[i]], sd).wait()
            return carry
        lax.fori_loop(0, PC, drain, 0)
    plsc.subcore_barrier()

    def wacc(t, carry):
        j = t * NS + sid

        @pl.when(j < N // ZA)
        def _():
            pltpu.sync_copy(acc.at[pl.ds(j * ZA, ZA)], r0b)
            pltpu.sync_copy(r0b, agg_hbm.at[cid, pl.ds(j * ZA, ZA)])
        return carry
    lax.fori_loop(0, -(-(N // ZA) // NS), wacc, 0)

    def wdg(t, carry):
        j = t * NS + sid

        @pl.when(j < N // ZD)
        def _():
            pltpu.sync_copy(deg.at[pl.ds(j * ZD, ZD)], zdeg)
            pltpu.sync_copy(zdeg, deg_hbm.at[cid, pl.ds(j * ZD, ZD)])
        return carry
    lax.fori_loop(0, -(-(N // ZD) // NS), wdg, 0)


_sc_aggregate = functools.partial(
    pl.kernel,
    out_type=(jax.ShapeDtypeStruct((NC, N, D), jnp.float32),
              jax.ShapeDtypeStruct((NC, N, DG), jnp.float32)),
    mesh=plsc.VectorSubcoreMesh(
        core_axis_name="c", subcore_axis_name="s",
        num_cores=NC, num_subcores=NS),
    scratch_types=(
        [pltpu.VMEM((PC, C), jnp.int32)] * 2
        + [pltpu.VMEM((C, D), jnp.float32)] * NB
        + [pltpu.VMEM((C, DG), jnp.float32),
           pltpu.VMEM((ZD, DG), jnp.float32),
           pltpu.VMEM_SHARED((N, D), jnp.float32),
           pltpu.VMEM_SHARED((N, DG), jnp.float32)]
        + [pltpu.SemaphoreType.DMA] * (NB + 1)
    ),
    compiler_params=pltpu.CompilerParams(use_tc_tiling_on_sc=False),
)(_sc_body)


def kernel(features, edge_index, W_mp, b_mp, W1, b1, W2, b2):
    parts, degp = _sc_aggregate(features, edge_index.reshape(2, NW, IT, C))

    out = pl.pallas_call(
        _head_body,
        out_shape=jax.ShapeDtypeStruct((N, D), jnp.float32),
    )(parts, degp, W_mp, b_mp.reshape(1, D), W1, b1.reshape(1, D),
      W2, b2.reshape(1, D))
    return out
